# Initial kernel scaffold; baseline (speedup 1.0000x reference)
import jax
import jax.numpy as jnp
from jax.experimental import pallas as pl


def kernel(x, time_day_table, time_week_table):
    raise NotImplementedError("write your pallas kernel here")


# speedup vs baseline: 2.1099x; 2.1099x over previous
import functools

import jax
import jax.numpy as jnp
from jax import lax
from jax.experimental import pallas as pl
from jax.experimental.pallas import tpu as pltpu
from jax.experimental.pallas import tpu_sc as plsc

_B, _T, _N, _F = 32, 12, 1024, 128
_D = 288
_W = 7
_HF = _F // 2
_L = 16
_M = _N * _T
_MC = 512
_NCHUNKS = _M // _MC


def _emb_body(xday_hbm, xweek_hbm, dayt_hbm, weekt_hbm, out_hbm,
              day_v, week_v, xd_v, xw_v, cd_v, cw_v, ob_v):
    b = lax.axis_index("s") * 2 + lax.axis_index("c")

    pltpu.sync_copy(dayt_hbm, day_v)
    pltpu.sync_copy(weekt_hbm, week_v)
    pltpu.sync_copy(xday_hbm.at[b], xd_v)
    pltpu.sync_copy(xweek_hbm.at[b], xw_v)

    lanes = lax.iota(jnp.int32, _L)

    def chunk_body(ci, _):
        m0 = ci * _MC

        def idx_body(g, _):
            m = m0 + g * _L + lanes
            tv = lax.rem(m, _T)
            nv = lax.div(m, _T)
            xdv = plsc.load_gather(xd_v, [tv, nv])
            xwv = plsc.load_gather(xw_v, [tv, nv])
            cd_v[pl.ds(g * _L, _L)] = (xdv * jnp.float32(_D)).astype(jnp.int32)
            cw_v[pl.ds(g * _L, _L)] = xwv.astype(jnp.int32)
            return 0

        lax.fori_loop(0, _MC // _L, idx_body, 0)

        def fill_g(g, _):
            dv = cd_v[pl.ds(g * _L, _L)]
            wv = cw_v[pl.ds(g * _L, _L)]

            def fill_f(f, _):
                fvec = jnp.full((_L,), f, jnp.int32)
                ob_v[f, pl.ds(g * _L, _L)] = plsc.load_gather(day_v, [dv, fvec])
                ob_v[f + _HF, pl.ds(g * _L, _L)] = plsc.load_gather(
                    week_v, [wv, fvec])
                return 0

            lax.fori_loop(0, _HF, fill_f, 0)
            return 0

        lax.fori_loop(0, _MC // _L, fill_g, 0)

        pltpu.sync_copy(ob_v, out_hbm.at[b, :, pl.ds(m0, _MC)])
        return 0

    lax.fori_loop(0, _NCHUNKS, chunk_body, 0)


_emb = functools.partial(
    pl.kernel,
    mesh=plsc.VectorSubcoreMesh(core_axis_name="c", subcore_axis_name="s"),
    out_type=jax.ShapeDtypeStruct((_B, _F, _M), jnp.float32),
    compiler_params=pltpu.CompilerParams(
        use_tc_tiling_on_sc=False, needs_layout_passes=False),
    scratch_types=[
        pltpu.VMEM((_D, _HF), jnp.float32),
        pltpu.VMEM((_W, _HF), jnp.float32),
        pltpu.VMEM((_T, _N), jnp.float32),
        pltpu.VMEM((_T, _N), jnp.float32),
        pltpu.VMEM((_MC,), jnp.int32),
        pltpu.VMEM((_MC,), jnp.int32),
        pltpu.VMEM((_F, _MC), jnp.float32),
    ],
)(_emb_body)


def kernel(x, time_day_table, time_week_table):
    xday = x[:, :, :, 1]
    xweek = x[:, :, :, 2]
    out3 = _emb(xday, xweek, time_day_table, time_week_table)
    return out3.reshape(_B, _F, _N, _T)


# speedup vs baseline: 2.1833x; 1.0348x over previous
import functools

import jax
import jax.numpy as jnp
from jax import lax
from jax.experimental import pallas as pl
from jax.experimental.pallas import tpu as pltpu
from jax.experimental.pallas import tpu_sc as plsc

_B, _T, _N, _F = 32, 12, 1024, 128
_D = 288
_W = 7
_HF = _F // 2
_L = 16
_M = _N * _T
_MC = 256
_NPAIR = _M // (2 * _MC)


def _emb_body(x_hbm, dayt_hbm, weekt_hbm, out_hbm,
              x_v, day_v, week_v, ob0, ob1, sem0, sem1):
    b = lax.axis_index("s") * 2 + lax.axis_index("c")

    pltpu.sync_copy(dayt_hbm, day_v)
    pltpu.sync_copy(weekt_hbm, week_v)
    pltpu.sync_copy(x_hbm.at[b], x_v)

    lanes = lax.iota(jnp.int32, _L)

    def fill(ob, m0):
        def fill_g(g, _):
            m = m0 + g * _L + lanes
            tv = lax.rem(m, _T)
            nv = lax.div(m, _T)
            base3 = tv * (3 * _N) + nv * 3
            xdv = plsc.load_gather(x_v, [base3 + 1])
            xwv = plsc.load_gather(x_v, [base3 + 2])
            dvf = (xdv * jnp.float32(_D)).astype(jnp.int32) * _HF
            wvf = xwv.astype(jnp.int32) * _HF
            col = g * _L
            for f in range(_HF):
                ob[f, pl.ds(col, _L)] = plsc.load_gather(day_v, [dvf])
                ob[f + _HF, pl.ds(col, _L)] = plsc.load_gather(week_v, [wvf])
                if f + 1 < _HF:
                    dvf = dvf + 1
                    wvf = wvf + 1
            return 0

        lax.fori_loop(0, _MC // _L, fill_g, 0)

    def pair_body(cp, _):
        for half, (ob, sem) in enumerate(((ob0, sem0), (ob1, sem1))):
            m0 = cp * (2 * _MC) + half * _MC

            @pl.when(cp > 0)
            def _wait_prev():
                pltpu.make_async_copy(
                    ob, out_hbm.at[b, :, pl.ds(m0 - 2 * _MC, _MC)], sem).wait()

            fill(ob, m0)
            pltpu.async_copy(ob, out_hbm.at[b, :, pl.ds(m0, _MC)], sem)
        return 0

    lax.fori_loop(0, _NPAIR, pair_body, 0)
    pltpu.make_async_copy(
        ob0, out_hbm.at[b, :, pl.ds(_M - 2 * _MC, _MC)], sem0).wait()
    pltpu.make_async_copy(
        ob1, out_hbm.at[b, :, pl.ds(_M - _MC, _MC)], sem1).wait()


_emb = functools.partial(
    pl.kernel,
    mesh=plsc.VectorSubcoreMesh(core_axis_name="c", subcore_axis_name="s"),
    out_type=jax.ShapeDtypeStruct((_B, _F, _M), jnp.float32),
    compiler_params=pltpu.CompilerParams(
        use_tc_tiling_on_sc=False, needs_layout_passes=False),
    scratch_types=[
        pltpu.VMEM((_T * _N * 3,), jnp.float32),
        pltpu.VMEM((_D * _HF,), jnp.float32),
        pltpu.VMEM((_W * _HF,), jnp.float32),
        pltpu.VMEM((_F, _MC), jnp.float32),
        pltpu.VMEM((_F, _MC), jnp.float32),
        pltpu.SemaphoreType.DMA,
        pltpu.SemaphoreType.DMA,
    ],
)(_emb_body)


def kernel(x, time_day_table, time_week_table):
    out3 = _emb(x.reshape(_B, _T * _N * 3),
                time_day_table.reshape(-1),
                time_week_table.reshape(-1))
    return out3.reshape(_B, _F, _N, _T)


# speedup vs baseline: 2.7718x; 1.2695x over previous
import functools

import jax
import jax.numpy as jnp
from jax import lax
from jax.experimental import pallas as pl
from jax.experimental.pallas import tpu as pltpu
from jax.experimental.pallas import tpu_sc as plsc

_B, _T, _N, _F = 32, 12, 1024, 128
_D = 288
_W = 7
_HF = _F // 2
_L = 16
_NC = 256
_NCHUNKS = _T * (_N // _NC)


def _emb_body(x_hbm, dayt_hbm, weekt_hbm, out_hbm,
              x_v, day_v, week_v, ob0, ob1, sem0, sem1):
    b = lax.axis_index("s") * 2 + lax.axis_index("c")

    pltpu.sync_copy(dayt_hbm, day_v)
    pltpu.sync_copy(weekt_hbm, week_v)
    pltpu.sync_copy(x_hbm.at[b], x_v)

    lanes = lax.iota(jnp.int32, _L)

    def fill(ob, t, n0):
        base_t = t * (3 * _N)

        def fill_g(g, _):
            nv = n0 + g * _L + lanes
            base3 = base_t + nv * 3
            xdv = plsc.load_gather(x_v, [base3 + 1])
            xwv = plsc.load_gather(x_v, [base3 + 2])
            dvf = (xdv * jnp.float32(_D)).astype(jnp.int32) * _HF
            wvf = xwv.astype(jnp.int32) * _HF
            col = g * _L
            for f in range(_HF):
                ob[f, pl.ds(col, _L)] = plsc.load_gather(day_v, [dvf])
                ob[f + _HF, pl.ds(col, _L)] = plsc.load_gather(week_v, [wvf])
                if f + 1 < _HF:
                    dvf = dvf + 1
                    wvf = wvf + 1
            return 0

        lax.fori_loop(0, _NC // _L, fill_g, 0)

    def pair_body(cp, _):
        for half, (ob, sem) in enumerate(((ob0, sem0), (ob1, sem1))):
            ci = cp * 2 + half
            t = lax.div(ci, _N // _NC)
            n0 = lax.rem(ci, _N // _NC) * _NC

            @pl.when(cp > 0)
            def _wait_prev():
                pltpu.make_async_copy(
                    ob, out_hbm.at[b, 0, :, pl.ds(0, _NC)], sem).wait()

            fill(ob, t, n0)
            pltpu.async_copy(ob, out_hbm.at[b, t, :, pl.ds(n0, _NC)], sem)
        return 0

    lax.fori_loop(0, _NCHUNKS // 2, pair_body, 0)
    pltpu.make_async_copy(ob0, out_hbm.at[b, 0, :, pl.ds(0, _NC)], sem0).wait()
    pltpu.make_async_copy(ob1, out_hbm.at[b, 0, :, pl.ds(0, _NC)], sem1).wait()


_emb = functools.partial(
    pl.kernel,
    mesh=plsc.VectorSubcoreMesh(core_axis_name="c", subcore_axis_name="s"),
    out_type=jax.ShapeDtypeStruct((_B, _T, _F, _N), jnp.float32),
    compiler_params=pltpu.CompilerParams(
        use_tc_tiling_on_sc=False, needs_layout_passes=False),
    scratch_types=[
        pltpu.VMEM((_T * _N * 3,), jnp.float32),
        pltpu.VMEM((_D * _HF,), jnp.float32),
        pltpu.VMEM((_W * _HF,), jnp.float32),
        pltpu.VMEM((_F, _NC), jnp.float32),
        pltpu.VMEM((_F, _NC), jnp.float32),
        pltpu.SemaphoreType.DMA,
        pltpu.SemaphoreType.DMA,
    ],
)(_emb_body)


def kernel(x, time_day_table, time_week_table):
    out4 = _emb(x.reshape(_B, _T * _N * 3),
                time_day_table.reshape(-1),
                time_week_table.reshape(-1))
    return jnp.transpose(out4, (0, 2, 3, 1))


# speedup vs baseline: 4.3401x; 1.5658x over previous
import functools

import jax
import jax.numpy as jnp
from jax import lax
from jax.experimental import pallas as pl
from jax.experimental.pallas import tpu as pltpu
from jax.experimental.pallas import tpu_sc as plsc

_B, _T, _N, _F = 32, 12, 1024, 128
_D = 288
_W = 7
_HF = _F // 2
_L = 16
_NC = 256
_NCHUNKS = _T * (_N // _NC)


def _emb_body(x_hbm, dayt_hbm, weekt_hbm, out_hbm,
              x_v, day_v, week_v, ob0, ob1, sem0, sem1):
    b = lax.axis_index("s") * 2 + lax.axis_index("c")

    pltpu.sync_copy(dayt_hbm, day_v)
    pltpu.sync_copy(weekt_hbm, week_v)
    pltpu.sync_copy(x_hbm.at[b], x_v)

    lanes = lax.iota(jnp.int32, _L)

    def fill(ob, t, n0):
        base_t = t * (3 * _N)

        def fill_g(g, _):
            nv = n0 + g * _L + lanes
            base3 = base_t + nv * 3
            xdv = plsc.load_gather(x_v, [base3 + 1])
            xwv = plsc.load_gather(x_v, [base3 + 2])
            dv0 = (xdv * jnp.float32(_D)).astype(jnp.int32) * _HF
            wv0 = xwv.astype(jnp.int32) * _HF
            col = g * _L

            @plsc.parallel_loop(0, _HF, 1, unroll=8)
            def _fill_f(f):
                ob[f, pl.ds(col, _L)] = plsc.load_gather(day_v, [dv0 + f])
                ob[f + _HF, pl.ds(col, _L)] = plsc.load_gather(
                    week_v, [wv0 + f])

            return 0

        lax.fori_loop(0, _NC // _L, fill_g, 0)

    def pair_body(cp, _):
        for half, (ob, sem) in enumerate(((ob0, sem0), (ob1, sem1))):
            ci = cp * 2 + half
            t = lax.div(ci, _N // _NC)
            n0 = lax.rem(ci, _N // _NC) * _NC

            @pl.when(cp > 0)
            def _wait_prev():
                pltpu.make_async_copy(
                    ob, out_hbm.at[b, 0, :, pl.ds(0, _NC)], sem).wait()

            fill(ob, t, n0)
            pltpu.async_copy(ob, out_hbm.at[b, t, :, pl.ds(n0, _NC)], sem)
        return 0

    lax.fori_loop(0, _NCHUNKS // 2, pair_body, 0)
    pltpu.make_async_copy(ob0, out_hbm.at[b, 0, :, pl.ds(0, _NC)], sem0).wait()
    pltpu.make_async_copy(ob1, out_hbm.at[b, 0, :, pl.ds(0, _NC)], sem1).wait()


_emb = functools.partial(
    pl.kernel,
    mesh=plsc.VectorSubcoreMesh(core_axis_name="c", subcore_axis_name="s"),
    out_type=jax.ShapeDtypeStruct((_B, _T, _F, _N), jnp.float32),
    compiler_params=pltpu.CompilerParams(
        use_tc_tiling_on_sc=False, needs_layout_passes=False),
    scratch_types=[
        pltpu.VMEM((_T * _N * 3,), jnp.float32),
        pltpu.VMEM((_D * _HF,), jnp.float32),
        pltpu.VMEM((_W * _HF,), jnp.float32),
        pltpu.VMEM((_F, _NC), jnp.float32),
        pltpu.VMEM((_F, _NC), jnp.float32),
        pltpu.SemaphoreType.DMA,
        pltpu.SemaphoreType.DMA,
    ],
)(_emb_body)


def kernel(x, time_day_table, time_week_table):
    out4 = _emb(x.reshape(_B, _T * _N * 3),
                time_day_table.reshape(-1),
                time_week_table.reshape(-1))
    return jnp.transpose(out4, (0, 2, 3, 1))


# speedup vs baseline: 6.1557x; 1.4183x over previous
import functools

import jax
import jax.numpy as jnp
from jax import lax
from jax.experimental import pallas as pl
from jax.experimental.pallas import tpu as pltpu
from jax.experimental.pallas import tpu_sc as plsc

_B, _T, _N, _F = 32, 12, 1024, 128
_D = 288
_W = 7
_HF = _F // 2
_L = 16
_NC = 256
_NCHUNKS = _T * (_N // _NC)


def _emb_body(x_hbm, dayt_hbm, weekt_hbm, out_hbm,
              x_v, day_v, week_v, ob0, ob1, sem0, sem1):
    b = lax.axis_index("s") * 2 + lax.axis_index("c")

    pltpu.sync_copy(dayt_hbm, day_v)
    pltpu.sync_copy(weekt_hbm, week_v)
    pltpu.sync_copy(x_hbm.at[b], x_v)

    lanes = lax.iota(jnp.int32, _L)

    def fill(ob, t, n0):
        base_t = t * (3 * _N)

        def fill_g(g, _):
            nv = n0 + g * _L + lanes
            base3 = base_t + nv * 3
            xdv = plsc.load_gather(x_v, [base3 + 1])
            xwv = plsc.load_gather(x_v, [base3 + 2])
            dv0 = (xdv * jnp.float32(_D)).astype(jnp.int32) * _HF
            wv0 = xwv.astype(jnp.int32) * _HF
            col = g * _L
            ch = col // 128
            cl = col % 128

            @plsc.parallel_loop(0, _HF, 1, unroll=8)
            def _fill_f(f):
                fa = lax.shift_right_logical(f, 3)
                fb = lax.rem(f, 8)
                ob[fa, ch, fb, pl.ds(cl, _L)] = plsc.load_gather(
                    day_v, [dv0 + f])
                ob[fa + _HF // 8, ch, fb, pl.ds(cl, _L)] = plsc.load_gather(
                    week_v, [wv0 + f])

            return 0

        lax.fori_loop(0, _NC // _L, fill_g, 0)

    _NT = _NC // 128

    def pair_body(cp, _):
        for half, (ob, sem) in enumerate(((ob0, sem0), (ob1, sem1))):
            ci = cp * 2 + half
            t = lax.div(ci, _N // _NC)
            n0 = lax.rem(ci, _N // _NC) * _NC
            na0 = lax.div(n0, 128)

            @pl.when(cp > 0)
            def _wait_prev():
                pltpu.make_async_copy(
                    ob, out_hbm.at[b, 0, :, pl.ds(0, _NT), :, :], sem).wait()

            fill(ob, t, n0)
            pltpu.async_copy(
                ob, out_hbm.at[b, t, :, pl.ds(na0, _NT), :, :], sem)
        return 0

    lax.fori_loop(0, _NCHUNKS // 2, pair_body, 0)
    pltpu.make_async_copy(
        ob0, out_hbm.at[b, 0, :, pl.ds(0, _NT), :, :], sem0).wait()
    pltpu.make_async_copy(
        ob1, out_hbm.at[b, 0, :, pl.ds(0, _NT), :, :], sem1).wait()


_emb = functools.partial(
    pl.kernel,
    mesh=plsc.VectorSubcoreMesh(core_axis_name="c", subcore_axis_name="s"),
    out_type=jax.ShapeDtypeStruct((_B, _T, _F // 8, _N // 128, 8, 128),
                                  jnp.float32),
    compiler_params=pltpu.CompilerParams(
        use_tc_tiling_on_sc=False, needs_layout_passes=False),
    scratch_types=[
        pltpu.VMEM((_T * _N * 3,), jnp.float32),
        pltpu.VMEM((_D * _HF,), jnp.float32),
        pltpu.VMEM((_W * _HF,), jnp.float32),
        pltpu.VMEM((_F // 8, _NC // 128, 8, 128), jnp.float32),
        pltpu.VMEM((_F // 8, _NC // 128, 8, 128), jnp.float32),
        pltpu.SemaphoreType.DMA,
        pltpu.SemaphoreType.DMA,
    ],
)(_emb_body)


def kernel(x, time_day_table, time_week_table):
    out6 = _emb(x.reshape(_B, _T * _N * 3),
                time_day_table.reshape(-1),
                time_week_table.reshape(-1))
    return jnp.transpose(out6, (0, 2, 4, 3, 5, 1)).reshape(_B, _F, _N, _T)


# speedup vs baseline: 16.5142x; 2.6828x over previous
import functools

import jax
import jax.numpy as jnp
from jax import lax
from jax.experimental import pallas as pl
from jax.experimental.pallas import tpu as pltpu
from jax.experimental.pallas import tpu_sc as plsc

_B, _T, _N, _F = 32, 12, 1024, 128
_D = 288
_W = 7
_HF = _F // 2
_L = 16
_NC = 256
_NCHUNKS = _T * (_N // _NC)


_DS = 65
_WS = 449


def _emb_body(x_hbm, dayt_hbm, weekt_hbm, out_hbm,
              x_v, day_v, week_v, ob0, ob1, sem0, sem1):
    b = lax.axis_index("s") * 2 + lax.axis_index("c")

    lanes = lax.iota(jnp.int32, _L)

    pltpu.sync_copy(dayt_hbm, x_v.at[pl.ds(0, _D * _HF)])

    @plsc.parallel_loop(0, _D * _HF // _L, 1, unroll=4)
    def _day_relayout(i):
        j = i * _L + lanes
        row = lax.div(j, _HF)
        colf = lax.rem(j, _HF)
        plsc.store_scatter(day_v, [row * _DS + colf], x_v[pl.ds(i * _L, _L)])

    pltpu.sync_copy(weekt_hbm, x_v.at[pl.ds(0, _W * _HF)])

    def _week_rep(l, _):
        def _week_chunk(i, _):
            j = i * _L + lanes
            plsc.store_scatter(week_v, [l * _WS + j], x_v[pl.ds(i * _L, _L)])
            return 0
        lax.fori_loop(0, _W * _HF // _L, _week_chunk, 0)
        return 0

    lax.fori_loop(0, _L, _week_rep, 0)

    pltpu.sync_copy(x_hbm.at[b], x_v)
    lane_ws = lanes * _WS

    def fill(ob, t, n0):
        base_t = t * (3 * _N)

        def fill_g(g, _):
            nv = n0 + g * _L + lanes
            base3 = base_t + nv * 3
            xdv = plsc.load_gather(x_v, [base3 + 1])
            xwv = plsc.load_gather(x_v, [base3 + 2])
            dv0 = (xdv * jnp.float32(_D)).astype(jnp.int32) * _DS
            wv0 = xwv.astype(jnp.int32) * _HF + lane_ws
            col = g * _L
            ch = col // 128
            cl = col % 128

            @plsc.parallel_loop(0, _HF, 1, unroll=8)
            def _fill_f(f):
                fa = lax.shift_right_logical(f, 3)
                fb = lax.rem(f, 8)
                ob[fa, ch, fb, pl.ds(cl, _L)] = plsc.load_gather(
                    day_v, [dv0 + f])
                ob[fa + _HF // 8, ch, fb, pl.ds(cl, _L)] = plsc.load_gather(
                    week_v, [wv0 + f])

            return 0

        lax.fori_loop(0, _NC // _L, fill_g, 0)

    _NT = _NC // 128

    def pair_body(cp, _):
        for half, (ob, sem) in enumerate(((ob0, sem0), (ob1, sem1))):
            ci = cp * 2 + half
            t = lax.div(ci, _N // _NC)
            n0 = lax.rem(ci, _N // _NC) * _NC
            na0 = lax.div(n0, 128)

            @pl.when(cp > 0)
            def _wait_prev():
                pltpu.make_async_copy(
                    ob, out_hbm.at[b, 0, :, pl.ds(0, _NT), :, :], sem).wait()

            fill(ob, t, n0)
            pltpu.async_copy(
                ob, out_hbm.at[b, t, :, pl.ds(na0, _NT), :, :], sem)
        return 0

    lax.fori_loop(0, _NCHUNKS // 2, pair_body, 0)
    pltpu.make_async_copy(
        ob0, out_hbm.at[b, 0, :, pl.ds(0, _NT), :, :], sem0).wait()
    pltpu.make_async_copy(
        ob1, out_hbm.at[b, 0, :, pl.ds(0, _NT), :, :], sem1).wait()


_emb = functools.partial(
    pl.kernel,
    mesh=plsc.VectorSubcoreMesh(core_axis_name="c", subcore_axis_name="s"),
    out_type=jax.ShapeDtypeStruct((_B, _T, _F // 8, _N // 128, 8, 128),
                                  jnp.float32),
    compiler_params=pltpu.CompilerParams(
        use_tc_tiling_on_sc=False, needs_layout_passes=False),
    scratch_types=[
        pltpu.VMEM((_T * _N * 3,), jnp.float32),
        pltpu.VMEM((_D * _DS,), jnp.float32),
        pltpu.VMEM((_L * _WS,), jnp.float32),
        pltpu.VMEM((_F // 8, _NC // 128, 8, 128), jnp.float32),
        pltpu.VMEM((_F // 8, _NC // 128, 8, 128), jnp.float32),
        pltpu.SemaphoreType.DMA,
        pltpu.SemaphoreType.DMA,
    ],
)(_emb_body)


def kernel(x, time_day_table, time_week_table):
    out6 = _emb(x.reshape(_B, _T * _N * 3),
                time_day_table.reshape(-1),
                time_week_table.reshape(-1))
    return jnp.transpose(out6, (0, 2, 4, 3, 5, 1)).reshape(_B, _F, _N, _T)


# speedup vs baseline: 25.7797x; 1.5611x over previous
import functools

import jax
import jax.numpy as jnp
from jax import lax
from jax.experimental import pallas as pl
from jax.experimental.pallas import tpu as pltpu
from jax.experimental.pallas import tpu_sc as plsc

_B, _T, _N, _F = 32, 12, 1024, 128
_D = 288
_W = 7
_HF = _F // 2
_L = 16
_NC = 256
_NCHUNKS = _T * (_N // _NC)
_DS = 65
_WS = 449
_TN = _T * _N


def _emb_body(xday_hbm, xweek_hbm, dayt_hbm, weekt_hbm, out_hbm,
              xd_v, xw_v, day_v, week_v, ob0, ob1, sem0, sem1):
    b = lax.axis_index("s") * 2 + lax.axis_index("c")

    pltpu.sync_copy(dayt_hbm, day_v)
    pltpu.sync_copy(weekt_hbm, week_v)
    pltpu.sync_copy(xday_hbm.at[pl.ds(b * _TN, _TN)], xd_v)
    pltpu.sync_copy(xweek_hbm.at[pl.ds(b * _TN, _TN)], xw_v)

    lanes = lax.iota(jnp.int32, _L)
    lane_ws = lanes * _WS

    def fill(ob, t, n0):
        base = t * _N + n0

        def fill_g(g, _):
            xdv = xd_v[pl.ds(base + g * _L, _L)]
            xwv = xw_v[pl.ds(base + g * _L, _L)]
            dv0 = (xdv * jnp.float32(_D)).astype(jnp.int32) * _DS
            wv0 = xwv.astype(jnp.int32) * _HF + lane_ws
            col = g * _L
            ch = col // 128
            cl = col % 128

            @plsc.parallel_loop(0, _HF, 1, unroll=8)
            def _fill_f(f):
                fa = lax.shift_right_logical(f, 3)
                fb = lax.rem(f, 8)
                ob[fa, ch, fb, pl.ds(cl, _L)] = plsc.load_gather(
                    day_v, [dv0 + f])
                ob[fa + _HF // 8, ch, fb, pl.ds(cl, _L)] = plsc.load_gather(
                    week_v, [wv0 + f])

            return 0

        lax.fori_loop(0, _NC // _L, fill_g, 0)

    _NT = _NC // 128

    def pair_body(cp, _):
        for half, (ob, sem) in enumerate(((ob0, sem0), (ob1, sem1))):
            ci = cp * 2 + half
            t = lax.div(ci, _N // _NC)
            n0 = lax.rem(ci, _N // _NC) * _NC
            na0 = lax.div(n0, 128)

            @pl.when(cp > 0)
            def _wait_prev():
                pltpu.make_async_copy(
                    ob, out_hbm.at[b, 0, :, pl.ds(0, _NT), :, :], sem).wait()

            fill(ob, t, n0)
            pltpu.async_copy(
                ob, out_hbm.at[b, t, :, pl.ds(na0, _NT), :, :], sem)
        return 0

    lax.fori_loop(0, _NCHUNKS // 2, pair_body, 0)
    pltpu.make_async_copy(
        ob0, out_hbm.at[b, 0, :, pl.ds(0, _NT), :, :], sem0).wait()
    pltpu.make_async_copy(
        ob1, out_hbm.at[b, 0, :, pl.ds(0, _NT), :, :], sem1).wait()


_emb = functools.partial(
    pl.kernel,
    mesh=plsc.VectorSubcoreMesh(core_axis_name="c", subcore_axis_name="s"),
    out_type=jax.ShapeDtypeStruct((_B, _T, _F // 8, _N // 128, 8, 128),
                                  jnp.float32),
    compiler_params=pltpu.CompilerParams(
        use_tc_tiling_on_sc=False, needs_layout_passes=False),
    scratch_types=[
        pltpu.VMEM((_TN,), jnp.float32),
        pltpu.VMEM((_TN,), jnp.float32),
        pltpu.VMEM((_D * _DS,), jnp.float32),
        pltpu.VMEM((_L * _WS,), jnp.float32),
        pltpu.VMEM((_F // 8, _NC // 128, 8, 128), jnp.float32),
        pltpu.VMEM((_F // 8, _NC // 128, 8, 128), jnp.float32),
        pltpu.SemaphoreType.DMA,
        pltpu.SemaphoreType.DMA,
    ],
)(_emb_body)


def kernel(x, time_day_table, time_week_table):
    xday = x[:, :, :, 1].reshape(-1)
    xweek = x[:, :, :, 2].reshape(-1)
    day_pad = jnp.pad(time_day_table, ((0, 0), (0, _DS - _HF))).reshape(-1)
    week_rep = jnp.pad(
        jnp.tile(time_week_table.reshape(1, _W * _HF), (_L, 1)),
        ((0, 0), (0, _WS - _W * _HF))).reshape(-1)
    out6 = _emb(xday, xweek, day_pad, week_rep)
    return jnp.transpose(out6, (0, 2, 4, 3, 5, 1)).reshape(_B, _F, _N, _T)
